# baseline (speedup 1.0000x reference)
import functools

import jax
import jax.numpy as jnp
import numpy as np
from jax import lax
from jax.experimental import pallas as pl
from jax.experimental.pallas import tpu as pltpu
from jax.experimental.pallas import tpu_sc as plsc

_N_BINS = 15
_BR = 32768


def _conf_body(n_rows, logits_ref, labels_ref, out_ref):
    x = logits_ref[...]
    c, b = x.shape
    m = jnp.max(x, axis=0, keepdims=True)
    e = jnp.exp(x - m)
    ind = jnp.where(e == 1.0, 1.0, 0.0)
    ones_row = jnp.ones((1, c), jnp.float32)
    iota_row = lax.broadcasted_iota(jnp.int32, (1, c), 1).astype(jnp.float32)
    dn = (((1,), (0,)), ((), ()))
    s = lax.dot_general(ones_row, e, dimension_numbers=dn,
                        preferred_element_type=jnp.float32)
    am = lax.dot_general(iota_row, ind, dimension_numbers=dn,
                         preferred_element_type=jnp.float32)
    conf = 1.0 / s
    labf = labels_ref[...].astype(jnp.float32).reshape(1, b)
    acc = am == labf
    signed = jnp.where(acc, conf, -conf)
    glob = pl.program_id(0) * b + lax.broadcasted_iota(jnp.int32, (1, b), 1)
    out_ref[...] = jnp.where(glob < n_rows, signed, 2.0).reshape(b)


def _tc_stage(logits, labels):
    n, c = logits.shape
    grid = (n + _BR - 1) // _BR
    n_pad = grid * _BR
    return pl.pallas_call(
        functools.partial(_conf_body, n),
        grid=(grid,),
        in_specs=[
            pl.BlockSpec((c, _BR), lambda i: (0, i)),
            pl.BlockSpec((_BR,), lambda i: (i,)),
        ],
        out_specs=pl.BlockSpec((_BR,), lambda i: (i,)),
        out_shape=jax.ShapeDtypeStruct((n_pad,), jnp.float32),
    )(logits.T, labels)


_NW = 16
_L = 16


def _sc_stage(sconf, n_real):
    n = sconf.shape[0]
    ch = n // _NW
    nvec = ch // _L
    assert ch * _NW == n and nvec * _L == ch and ch % 8 == 0

    uppers = [float(np.float32(i) / np.float32(_N_BINS))
              for i in range(1, _N_BINS)] + [1.0]
    inv_n = 1.0 / n_real
    unroll = 4
    assert nvec % unroll == 0 and nvec < 4096

    mesh = plsc.VectorSubcoreMesh(core_axis_name="c", subcore_axis_name="s",
                                  num_cores=1)

    @functools.partial(
        pl.kernel,
        mesh=mesh,
        out_type=jax.ShapeDtypeStruct((_L,), jnp.float32),
        compiler_params=pltpu.CompilerParams(needs_layout_passes=False),
        scratch_types=[
            pltpu.VMEM((ch,), jnp.float32),
            pltpu.VMEM((unroll * 2 * 16 * _L,), jnp.float32),
            pltpu.VMEM((3 * 16 * _L,), jnp.float32),
            pltpu.VMEM((_NW, 3 * 16 * _L), jnp.float32),
            pltpu.VMEM((_L,), jnp.float32),
            pltpu.VMEM_SHARED((_NW, 3 * 16 * _L), jnp.float32),
            pltpu.SemaphoreType.DMA,
            pltpu.SemaphoreType.DMA,
            pltpu.SemaphoreType.DMA,
            pltpu.SemaphoreType.DMA,
        ],
    )
    def sc_kernel(sconf_hbm, out_hbm, chunk_v, bank_v, part_v, comb_v, outv_v,
                  shared, sem0, sem1, sem2, sem3):
        wid = lax.axis_index("s")
        lane = lax.iota(jnp.int32, _L)
        zeros = jnp.zeros((_L,), jnp.float32)
        one_i = jnp.ones((_L,), jnp.int32)
        zero_i = jnp.zeros((_L,), jnp.int32)

        def zk(k, _):
            bank_v[pl.ds(k * _L, _L)] = zeros
            return 0
        lax.fori_loop(0, unroll * 2 * 16, zk, 0)

        nsub = 4
        sub = ch // nsub
        sems = [sem0, sem1, sem2, sem3]
        cps = []
        for p in range(nsub):
            cp = pltpu.make_async_copy(
                sconf_hbm.at[pl.ds(wid * ch + p * sub, sub)],
                chunk_v.at[pl.ds(p * sub, sub)], sems[p])
            cp.start()
            cps.append(cp)

        def one_vec(t, j):
            v = chunk_v[pl.ds(t * _L, _L)]
            conf = jnp.abs(v)
            packed = jnp.where(v > 0, 4097.0, 1.0)
            idx = zero_i
            for u in uppers:
                idx = idx + jnp.where(conf > u, one_i, zero_i)
            off = j * 512 + idx * _L + lane
            plsc.addupdate_scatter(bank_v, [off], packed)
            plsc.addupdate_scatter(bank_v, [off + 256], conf)

        def body(t, _):
            for j in range(unroll):
                one_vec(t * unroll + j, j)
            return 0
        nbody = nvec // unroll
        assert nbody % nsub == 0
        for p in range(nsub):
            cps[p].wait()
            lax.fori_loop(p * (nbody // nsub), (p + 1) * (nbody // nsub),
                          body, 0)

        def fold_k(k, _):
            def add_j(j, t):
                return t + bank_v[pl.ds(j * 512 + k * _L, _L)]
            tot = lax.fori_loop(1, unroll, add_j, bank_v[pl.ds(k * _L, _L)])
            bank_v[pl.ds(k * _L, _L)] = tot
            return 0
        lax.fori_loop(0, 2 * 16, fold_k, 0)

        def unpack_k(k, _):
            cell = bank_v[pl.ds(k * _L, _L)]
            q = (cell * (1.0 / 4096.0)).astype(jnp.int32).astype(jnp.float32)
            part_v[pl.ds(k * _L, _L)] = cell - 4096.0 * q
            part_v[pl.ds(512 + k * _L, _L)] = q
            part_v[pl.ds(256 + k * _L, _L)] = bank_v[pl.ds(256 + k * _L, _L)]
            return 0
        lax.fori_loop(0, 16, unpack_k, 0)

        pltpu.sync_copy(part_v, shared.at[wid])
        plsc.subcore_barrier()

        @pl.when(wid == 0)
        def _():
            pltpu.sync_copy(shared, comb_v)

            def comb_k(k, _):
                def add_w(w, t):
                    return t + comb_v[w, pl.ds(k * _L, _L)]
                tot = lax.fori_loop(1, _NW, add_w, comb_v[0, pl.ds(k * _L, _L)])
                part_v[pl.ds(k * _L, _L)] = tot
                return 0
            lax.fori_loop(0, 3 * 16, comb_k, 0)

            vecs = []
            for a in range(3):
                vec = zeros
                for b in range(16):
                    s = jnp.sum(part_v[pl.ds(a * 256 + b * _L, _L)])
                    vec = jnp.where(lane == b, s, vec)
                vecs.append(vec)
            cntv, confv, accv = vecs
            denom = jnp.maximum(cntv, 1.0)
            avg_c = confv / denom
            avg_a = accv / denom
            prop = cntv * inv_n
            contrib = jnp.where(cntv > 0.0,
                                jnp.abs(avg_c - avg_a) * prop, 0.0)
            contrib = jnp.where(lane < _N_BINS, contrib, 0.0)
            outv_v[...] = jnp.sum(contrib) + zeros
            pltpu.sync_copy(outv_v, out_hbm)

    return sc_kernel(sconf)


def kernel(logits, labels):
    sconf = _tc_stage(logits, labels)
    out = _sc_stage(sconf, logits.shape[0])
    return out[:1]


# speedup vs baseline: 1.0752x; 1.0752x over previous
import functools

import jax
import jax.numpy as jnp
import numpy as np
from jax import lax
from jax.experimental import pallas as pl
from jax.experimental.pallas import tpu as pltpu
from jax.experimental.pallas import tpu_sc as plsc

_N_BINS = 15
_BR = 32768


def _conf_body(n_rows, logits_ref, labels_ref, out_ref):
    x = logits_ref[...]
    c, b = x.shape
    m = jnp.max(x, axis=0, keepdims=True)
    e = jnp.exp(x - m)
    ind = jnp.where(e == 1.0, 1.0, 0.0)
    ones_row = jnp.ones((1, c), jnp.float32)
    iota_row = lax.broadcasted_iota(jnp.int32, (1, c), 1).astype(jnp.float32)
    dn = (((1,), (0,)), ((), ()))
    s = lax.dot_general(ones_row, e, dimension_numbers=dn,
                        preferred_element_type=jnp.float32)
    am = lax.dot_general(iota_row, ind, dimension_numbers=dn,
                         preferred_element_type=jnp.float32)
    conf = 1.0 / s
    labf = labels_ref[...].astype(jnp.float32).reshape(1, b)
    acc = am == labf
    signed = jnp.where(acc, conf, -conf)
    glob = pl.program_id(0) * b + lax.broadcasted_iota(jnp.int32, (1, b), 1)
    out_ref[...] = jnp.where(glob < n_rows, signed, 2.0).reshape(b)


def _tc_stage(logits, labels):
    n, c = logits.shape
    grid = (n + _BR - 1) // _BR
    n_pad = grid * _BR
    return pl.pallas_call(
        functools.partial(_conf_body, n),
        grid=(grid,),
        in_specs=[
            pl.BlockSpec((c, _BR), lambda i: (0, i)),
            pl.BlockSpec((_BR,), lambda i: (i,)),
        ],
        out_specs=pl.BlockSpec((_BR,), lambda i: (i,)),
        out_shape=jax.ShapeDtypeStruct((n_pad,), jnp.float32),
    )(logits.T, labels)


_NW = 16
_L = 16


def _sc_stage(sconf, n_real):
    n = sconf.shape[0]
    ch = n // _NW
    nvec = ch // _L
    assert ch * _NW == n and nvec * _L == ch and ch % 8 == 0

    uppers = [float(np.float32(i) / np.float32(_N_BINS))
              for i in range(1, _N_BINS)] + [1.0]
    inv_n = 1.0 / n_real
    unroll = 4
    assert nvec % unroll == 0 and nvec < 4096

    mesh = plsc.VectorSubcoreMesh(core_axis_name="c", subcore_axis_name="s",
                                  num_cores=1)

    @functools.partial(
        pl.kernel,
        mesh=mesh,
        out_type=jax.ShapeDtypeStruct((_L,), jnp.float32),
        compiler_params=pltpu.CompilerParams(needs_layout_passes=False),
        scratch_types=[
            pltpu.VMEM((ch,), jnp.float32),
            pltpu.VMEM((unroll * 2 * 16 * _L,), jnp.float32),
            pltpu.VMEM((3 * 16 * _L,), jnp.float32),
            pltpu.VMEM((_NW, 3 * 16 * _L), jnp.float32),
            pltpu.VMEM((_L,), jnp.float32),
            pltpu.VMEM_SHARED((_NW, 3 * 16 * _L), jnp.float32),
            pltpu.SemaphoreType.DMA,
            pltpu.SemaphoreType.DMA,
            pltpu.SemaphoreType.DMA,
            pltpu.SemaphoreType.DMA,
        ],
    )
    def sc_kernel(sconf_hbm, out_hbm, chunk_v, bank_v, part_v, comb_v, outv_v,
                  shared, sem0, sem1, sem2, sem3):
        wid = lax.axis_index("s")
        lane = lax.iota(jnp.int32, _L)
        zeros = jnp.zeros((_L,), jnp.float32)
        one_i = jnp.ones((_L,), jnp.int32)
        zero_i = jnp.zeros((_L,), jnp.int32)
        full14_i = jnp.full((_L,), 14, jnp.int32)
        full15_i = jnp.full((_L,), 15, jnp.int32)

        def zk(k, _):
            bank_v[pl.ds(k * _L, _L)] = zeros
            return 0
        lax.fori_loop(0, unroll * 2 * 16, zk, 0)

        nsub = 4
        sub = ch // nsub
        sems = [sem0, sem1, sem2, sem3]
        cps = []
        for p in range(nsub):
            cp = pltpu.make_async_copy(
                sconf_hbm.at[pl.ds(wid * ch + p * sub, sub)],
                chunk_v.at[pl.ds(p * sub, sub)], sems[p])
            cp.start()
            cps.append(cp)

        def one_vec(t, j):
            v = chunk_v[pl.ds(t * _L, _L)]
            conf = jnp.abs(v)
            packed = jnp.where(v > 0, 4097.0, 1.0)
            idx = jnp.minimum((conf * 15.0).astype(jnp.int32), full14_i)
            idx = jnp.where(conf > 1.0, full15_i, idx)
            off = j * 512 + idx * _L + lane
            plsc.addupdate_scatter(bank_v, [off], packed)
            plsc.addupdate_scatter(bank_v, [off + 256], conf)

        def body(t, _):
            for j in range(unroll):
                one_vec(t * unroll + j, j)
            return 0
        nbody = nvec // unroll
        assert nbody % nsub == 0
        for p in range(nsub):
            cps[p].wait()
            lax.fori_loop(p * (nbody // nsub), (p + 1) * (nbody // nsub),
                          body, 0)

        def fold_k(k, _):
            def add_j(j, t):
                return t + bank_v[pl.ds(j * 512 + k * _L, _L)]
            tot = lax.fori_loop(1, unroll, add_j, bank_v[pl.ds(k * _L, _L)])
            bank_v[pl.ds(k * _L, _L)] = tot
            return 0
        lax.fori_loop(0, 2 * 16, fold_k, 0)

        def unpack_k(k, _):
            cell = bank_v[pl.ds(k * _L, _L)]
            q = (cell * (1.0 / 4096.0)).astype(jnp.int32).astype(jnp.float32)
            part_v[pl.ds(k * _L, _L)] = cell - 4096.0 * q
            part_v[pl.ds(512 + k * _L, _L)] = q
            part_v[pl.ds(256 + k * _L, _L)] = bank_v[pl.ds(256 + k * _L, _L)]
            return 0
        lax.fori_loop(0, 16, unpack_k, 0)

        pltpu.sync_copy(part_v, shared.at[wid])
        plsc.subcore_barrier()

        @pl.when(wid == 0)
        def _():
            pltpu.sync_copy(shared, comb_v)

            def comb_k(k, _):
                def add_w(w, t):
                    return t + comb_v[w, pl.ds(k * _L, _L)]
                tot = lax.fori_loop(1, _NW, add_w, comb_v[0, pl.ds(k * _L, _L)])
                part_v[pl.ds(k * _L, _L)] = tot
                return 0
            lax.fori_loop(0, 3 * 16, comb_k, 0)

            vecs = []
            for a in range(3):
                vec = zeros
                for b in range(16):
                    s = jnp.sum(part_v[pl.ds(a * 256 + b * _L, _L)])
                    vec = jnp.where(lane == b, s, vec)
                vecs.append(vec)
            cntv, confv, accv = vecs
            denom = jnp.maximum(cntv, 1.0)
            avg_c = confv / denom
            avg_a = accv / denom
            prop = cntv * inv_n
            contrib = jnp.where(cntv > 0.0,
                                jnp.abs(avg_c - avg_a) * prop, 0.0)
            contrib = jnp.where(lane < _N_BINS, contrib, 0.0)
            outv_v[...] = jnp.sum(contrib) + zeros
            pltpu.sync_copy(outv_v, out_hbm)

    return sc_kernel(sconf)


def kernel(logits, labels):
    sconf = _tc_stage(logits, labels)
    out = _sc_stage(sconf, logits.shape[0])
    return out[:1]
